# baseline (speedup 1.0000x reference)
import functools

import jax
import jax.numpy as jnp
from jax import lax
from jax.experimental import pallas as pl
from jax.experimental.pallas import tpu as pltpu
from jax.experimental.pallas import tpu_sc as plsc

N = 10000
E = 320000
D = 128
DE = 16
G = 64

NC = 2
NS = 16
NW = NC * NS
EPW = E // NW
CH = 640
SEG = 64
STAGE = SEG + 16

NPAD = 10240
ROWS_PT = 640
TRASH = ROWS_PT
ACC_ROWS = ROWS_PT + 8


@functools.cache
def _sc_mesh():
    return plsc.VectorSubcoreMesh(core_axis_name="c", subcore_axis_name="s",
                                  num_cores=NC, num_subcores=NS)


def _consume_body(hw_hbm, src_hbm, dst_hbm, attr_hbm, pout_hbm, aout_hbm,
                  srcc_v, dstc_v, attrc_v, ssrc_v, sdl_v, sattr_v, rows_v,
                  acc_v, aacc_v, cnt_sm, sem):
    c = lax.axis_index("c")
    me = lax.axis_index("s")
    zf = jnp.zeros((16,), jnp.float32)
    iota16 = lax.iota(jnp.int32, 16)
    trash16 = jnp.zeros((16,), jnp.int32) + TRASH
    zero16i = jnp.zeros((16,), jnp.int32)

    def zrow(i, carry):
        for k8 in range(D // 16):
            acc_v[i, pl.ds(k8 * 16, 16)] = zf
        aacc_v[pl.ds(i * DE, 16)] = zf
        return carry

    lax.fori_loop(0, ACC_ROWS, zrow, 0)
    cnt_sm[0] = 0

    def process64():
        pltpu.async_copy(hw_hbm.at[ssrc_v.at[pl.ds(0, SEG)]], rows_v, sem).wait()

        def grp(q, carry3):
            d16 = sdl_v[pl.ds(q * 16, 16)]
            for l in range(16):
                dl = d16[l]
                j = q * 16 + l
                for k8 in range(D // 16):
                    sl = pl.ds(k8 * 16, 16)
                    acc_v[dl, sl] = acc_v[dl, sl] + rows_v[j, sl]
                asl = pl.ds(dl * DE, 16)
                aacc_v[asl] = aacc_v[asl] + sattr_v[pl.ds(j * DE, 16)]
            return carry3

        lax.fori_loop(0, SEG // 16, grp, 0)

    def pad_stage(cnt):
        for g in range(STAGE // 16):
            posv = iota16 + (g * 16)
            keep = posv < cnt
            sdl_v[pl.ds(g * 16, 16)] = jnp.where(keep, sdl_v[pl.ds(g * 16, 16)],
                                                 trash16)
            ssrc_v[pl.ds(g * 16, 16)] = jnp.where(keep, ssrc_v[pl.ds(g * 16, 16)],
                                                  zero16i)

        def zattr(q, carry):
            @pl.when(q >= cnt)
            def _():
                sattr_v[pl.ds(q * DE, 16)] = zf
            return carry

        lax.fori_loop(0, STAGE, zattr, 0)

    def chunk(k, carry):
        eb = pl.multiple_of(c * (E // NC) + k * CH, 8)
        pltpu.sync_copy(src_hbm.at[pl.ds(eb, CH)], srcc_v)
        pltpu.sync_copy(dst_hbm.at[pl.ds(eb, CH)], dstc_v)
        pltpu.sync_copy(attr_hbm.at[pl.ds(eb * DE, CH * DE)], attrc_v)

        def grp(j, carry2):
            d16 = dstc_v[pl.ds(j * 16, 16)]
            s16 = srcc_v[pl.ds(j * 16, 16)]
            o16 = lax.shift_right_logical(d16, 5) & 15
            dl16 = lax.shift_left(lax.shift_right_logical(d16, 9), 5) | (d16 & 31)
            m = o16 == me
            incl = plsc.cumsum(jnp.where(m, 1, 0))
            r16 = incl - jnp.where(m, 1, 0)
            tot = incl[15]
            cnt = cnt_sm[0]
            pos16 = r16 + cnt
            plsc.store_scatter(ssrc_v, [pos16], s16, mask=m)
            plsc.store_scatter(sdl_v, [pos16], dl16, mask=m)
            for col in range(DE):
                a16 = plsc.load_gather(attrc_v, [iota16 * DE + (j * 16 * DE + col)])
                plsc.store_scatter(sattr_v, [pos16 * DE + col], a16, mask=m)
            cnt2 = cnt + tot
            cnt_sm[0] = cnt2

            @pl.when(cnt2 >= SEG)
            def _():
                process64()
                sdl_v[pl.ds(0, 16)] = sdl_v[pl.ds(SEG, 16)]
                ssrc_v[pl.ds(0, 16)] = ssrc_v[pl.ds(SEG, 16)]
                for q in range(STAGE - SEG):
                    sattr_v[pl.ds(q * DE, 16)] = sattr_v[pl.ds((q + SEG) * DE, 16)]
                cnt_sm[0] = cnt2 - SEG
            return carry2

        lax.fori_loop(0, CH // 16, grp, 0)
        return carry

    lax.fori_loop(0, (E // NC) // CH, chunk, 0)

    cnt = cnt_sm[0]
    pad_stage(cnt)
    process64()

    for b in range(ROWS_PT // 32):
        gbase = b << 9
        pltpu.sync_copy(
            acc_v.at[pl.ds(b * 32, 32)],
            pout_hbm.at[c, pl.ds(pl.multiple_of(gbase + me * 32, 32), 32)])
        pltpu.sync_copy(
            aacc_v.at[pl.ds(b * 32 * DE, 32 * DE)],
            aout_hbm.at[c, pl.ds(pl.multiple_of((gbase + me * 32) * DE, 32 * DE),
                                 32 * DE)])


@functools.cache
def _sc_consume():
    return pl.kernel(
        _consume_body,
        out_type=(jax.ShapeDtypeStruct((NC, NPAD, D), jnp.float32),
                  jax.ShapeDtypeStruct((NC, NPAD * DE), jnp.float32)),
        mesh=_sc_mesh(),
        compiler_params=pltpu.CompilerParams(needs_layout_passes=False),
        scratch_types=[
            pltpu.VMEM((CH,), jnp.int32),
            pltpu.VMEM((CH,), jnp.int32),
            pltpu.VMEM((CH * DE,), jnp.float32),
            pltpu.VMEM((STAGE,), jnp.int32),
            pltpu.VMEM((STAGE,), jnp.int32),
            pltpu.VMEM((STAGE * DE,), jnp.float32),
            pltpu.VMEM((SEG, D), jnp.float32),
            pltpu.VMEM((ACC_ROWS, D), jnp.float32),
            pltpu.VMEM((ACC_ROWS * DE,), jnp.float32),
            pltpu.SMEM((8,), jnp.int32),
            pltpu.SemaphoreType.DMA,
        ],
    )


def _dense_body(x_ref, w_ref, b_ref, o_ref):
    o_ref[...] = (jnp.dot(x_ref[...], w_ref[...], preferred_element_type=jnp.float32)
                  + b_ref[...])


def _dense(x, w, b):
    return pl.pallas_call(
        _dense_body,
        out_shape=jax.ShapeDtypeStruct((x.shape[0], w.shape[1]), jnp.float32),
    )(x, w, b[None, :])


def _graph_norm_block(agg, batch2, gnw, gnb, gnms):
    onehot = (batch2 == lax.broadcasted_iota(jnp.int32, (1, G), 1)).astype(jnp.float32)
    counts = jnp.maximum(jnp.sum(onehot, axis=0, keepdims=True), 1.0).reshape(G, 1)
    sums = lax.dot_general(onehot, agg, (((0,), (0,)), ((), ())),
                           preferred_element_type=jnp.float32)
    mean_s = (sums / counts) * gnms
    centered = agg - jnp.dot(onehot, mean_s, preferred_element_type=jnp.float32)
    var = lax.dot_general(onehot, centered * centered, (((0,), (0,)), ((), ())),
                          preferred_element_type=jnp.float32) / counts
    inv = 1.0 / jnp.sqrt(var + 1e-5)
    y = gnw * centered * jnp.dot(onehot, inv, preferred_element_type=jnp.float32) + gnb
    return jnp.where(y >= 0.0, y, 0.01 * y)


def _norm_mid_body(p_ref, a_ref, h_ref, we_ref, batch_ref,
                   gnw_ref, gnb_ref, gnms_ref, wm_ref, bm_ref, o_ref, ohw_ref):
    asum = a_ref[0] + a_ref[1]
    agg = (p_ref[0] + p_ref[1]
           + jnp.dot(asum, we_ref[...], preferred_element_type=jnp.float32)
           + h_ref[...])
    y = _graph_norm_block(agg, batch_ref[...], gnw_ref[...], gnb_ref[...], gnms_ref[...])
    o_ref[...] = y
    ohw_ref[...] = (jnp.dot(y, wm_ref[...], preferred_element_type=jnp.float32)
                    + bm_ref[...])


def _norm_fin_body(p_ref, a_ref, h_ref, we_ref, batch_ref,
                   gnw_ref, gnb_ref, gnms_ref, o_ref):
    asum = a_ref[0] + a_ref[1]
    agg = (p_ref[0] + p_ref[1]
           + jnp.dot(asum, we_ref[...], preferred_element_type=jnp.float32)
           + h_ref[...])
    o_ref[...] = _graph_norm_block(agg, batch_ref[...], gnw_ref[...], gnb_ref[...],
                                   gnms_ref[...])


def kernel(x, index, attr, batch, W_msg0, b_msg0, W_edge0, b_edge0, gn_w0, gn_b0,
           gn_ms0, W_msg1, b_msg1, W_edge1, b_edge1, gn_w1, gn_b1, gn_ms1):
    src = index[0]
    dst = index[1]
    batch2 = batch[:, None]

    attr_flat = attr.reshape(E * DE)
    hw0 = _dense(x, W_msg0, b_msg0 + b_edge0)
    p0, a0 = _sc_consume()(hw0, src, dst, attr_flat)
    p0 = p0[:, :N]
    a0 = a0.reshape(NC, NPAD, DE)[:, :N]
    h1, hw1 = pl.pallas_call(
        _norm_mid_body,
        out_shape=(jax.ShapeDtypeStruct((N, D), jnp.float32),
                   jax.ShapeDtypeStruct((N, D), jnp.float32)),
    )(p0, a0, x, W_edge0, batch2, gn_w0[None, :], gn_b0[None, :], gn_ms0[None, :],
      W_msg1, (b_msg1 + b_edge1)[None, :])
    p1 = _sc_consume()(hw1, src, dst, attr_flat)[0][:, :N]
    h2 = pl.pallas_call(
        _norm_fin_body,
        out_shape=jax.ShapeDtypeStruct((N, D), jnp.float32),
    )(p1, a0, h1, W_edge1, batch2, gn_w1[None, :], gn_b1[None, :], gn_ms1[None, :])
    return h2


# speedup vs baseline: 1.3630x; 1.3630x over previous
import functools

import jax
import jax.numpy as jnp
from jax import lax
from jax.experimental import pallas as pl
from jax.experimental.pallas import tpu as pltpu
from jax.experimental.pallas import tpu_sc as plsc

N = 10000
E = 320000
D = 128
DE = 16
G = 64

NC = 2
NS = 16
NW = NC * NS
EPW = E // NW
CH = 640
SEG = 64
STAGE = SEG + 16

NPAD = 10240
ROWS_PT = 640
TRASH = ROWS_PT
ACC_ROWS = ROWS_PT + 8


@functools.cache
def _sc_mesh():
    return plsc.VectorSubcoreMesh(core_axis_name="c", subcore_axis_name="s",
                                  num_cores=NC, num_subcores=NS)


def _consume_body(with_attr, hw_hbm, src_hbm, dst_hbm, attr_hbm, pout_hbm,
                  aout_hbm, srcc_v, dstc_v, attrc_v, ssrc_v, sdl_v, sattr_v,
                  rows_v, acc_v, aacc_v, cnt_sm, sem):
    c = lax.axis_index("c")
    me = lax.axis_index("s")
    zf = jnp.zeros((16,), jnp.float32)
    iota16 = lax.iota(jnp.int32, 16)
    trash16 = jnp.zeros((16,), jnp.int32) + TRASH
    zero16i = jnp.zeros((16,), jnp.int32)

    def zrow(i, carry):
        for k8 in range(D // 16):
            acc_v[i, pl.ds(k8 * 16, 16)] = zf
        if with_attr:
            aacc_v[pl.ds(i * DE, 16)] = zf
        return carry

    lax.fori_loop(0, ACC_ROWS, zrow, 0)
    cnt_sm[0] = 0

    def process64():
        pltpu.async_copy(hw_hbm.at[ssrc_v.at[pl.ds(0, SEG)]], rows_v, sem).wait()

        def grp(q, carry3):
            d16 = sdl_v[pl.ds(q * 16, 16)]
            for l in range(16):
                dl = d16[l]
                j = q * 16 + l
                for k8 in range(D // 16):
                    sl = pl.ds(k8 * 16, 16)
                    acc_v[dl, sl] = acc_v[dl, sl] + rows_v[j, sl]
                if with_attr:
                    asl = pl.ds(dl * DE, 16)
                    aacc_v[asl] = aacc_v[asl] + sattr_v[pl.ds(j * DE, 16)]
            return carry3

        lax.fori_loop(0, SEG // 16, grp, 0)

    def pad_stage(cnt):
        for g in range(STAGE // 16):
            posv = iota16 + (g * 16)
            keep = posv < cnt
            sdl_v[pl.ds(g * 16, 16)] = jnp.where(keep, sdl_v[pl.ds(g * 16, 16)],
                                                 trash16)
            ssrc_v[pl.ds(g * 16, 16)] = jnp.where(keep, ssrc_v[pl.ds(g * 16, 16)],
                                                  zero16i)

        if with_attr:
            def zattr(q, carry):
                @pl.when(q >= cnt)
                def _():
                    sattr_v[pl.ds(q * DE, 16)] = zf
                return carry

            lax.fori_loop(0, STAGE, zattr, 0)

    def chunk(k, carry):
        eb = pl.multiple_of(c * (E // NC) + k * CH, 8)
        pltpu.sync_copy(src_hbm.at[pl.ds(eb, CH)], srcc_v)
        pltpu.sync_copy(dst_hbm.at[pl.ds(eb, CH)], dstc_v)
        if with_attr:
            pltpu.sync_copy(attr_hbm.at[pl.ds(eb * DE, CH * DE)], attrc_v)

        def grp(j, carry2):
            d16 = dstc_v[pl.ds(j * 16, 16)]
            s16 = srcc_v[pl.ds(j * 16, 16)]
            o16 = lax.shift_right_logical(d16, 5) & 15
            dl16 = lax.shift_left(lax.shift_right_logical(d16, 9), 5) | (d16 & 31)
            m = o16 == me
            incl = plsc.cumsum(jnp.where(m, 1, 0))
            r16 = incl - jnp.where(m, 1, 0)
            tot = incl[15]
            cnt = cnt_sm[0]
            pos16 = r16 + cnt
            plsc.store_scatter(ssrc_v, [pos16], s16, mask=m)
            plsc.store_scatter(sdl_v, [pos16], dl16, mask=m)
            if with_attr:
                for col in range(DE):
                    a16 = plsc.load_gather(attrc_v,
                                           [iota16 * DE + (j * 16 * DE + col)])
                    plsc.store_scatter(sattr_v, [pos16 * DE + col], a16, mask=m)
            cnt2 = cnt + tot
            cnt_sm[0] = cnt2

            @pl.when(cnt2 >= SEG)
            def _():
                process64()
                sdl_v[pl.ds(0, 16)] = sdl_v[pl.ds(SEG, 16)]
                ssrc_v[pl.ds(0, 16)] = ssrc_v[pl.ds(SEG, 16)]
                if with_attr:
                    for q in range(STAGE - SEG):
                        sattr_v[pl.ds(q * DE, 16)] = sattr_v[
                            pl.ds((q + SEG) * DE, 16)]
                cnt_sm[0] = cnt2 - SEG
            return carry2

        lax.fori_loop(0, CH // 16, grp, 0)
        return carry

    lax.fori_loop(0, (E // NC) // CH, chunk, 0)

    cnt = cnt_sm[0]
    pad_stage(cnt)
    process64()

    for b in range(ROWS_PT // 32):
        gbase = b << 9
        pltpu.sync_copy(
            acc_v.at[pl.ds(b * 32, 32)],
            pout_hbm.at[c, pl.ds(pl.multiple_of(gbase + me * 32, 32), 32)])
        if with_attr:
            pltpu.sync_copy(
                aacc_v.at[pl.ds(b * 32 * DE, 32 * DE)],
                aout_hbm.at[c,
                            pl.ds(pl.multiple_of((gbase + me * 32) * DE, 32 * DE),
                                  32 * DE)])


@functools.cache
def _sc_consume(with_attr):
    return pl.kernel(
        functools.partial(_consume_body, with_attr),
        out_type=(jax.ShapeDtypeStruct((NC, NPAD, D), jnp.float32),
                  jax.ShapeDtypeStruct((NC, NPAD * DE), jnp.float32)),
        mesh=_sc_mesh(),
        compiler_params=pltpu.CompilerParams(needs_layout_passes=False),
        scratch_types=[
            pltpu.VMEM((CH,), jnp.int32),
            pltpu.VMEM((CH,), jnp.int32),
            pltpu.VMEM((CH * DE,), jnp.float32),
            pltpu.VMEM((STAGE,), jnp.int32),
            pltpu.VMEM((STAGE,), jnp.int32),
            pltpu.VMEM((STAGE * DE,), jnp.float32),
            pltpu.VMEM((SEG, D), jnp.float32),
            pltpu.VMEM((ACC_ROWS, D), jnp.float32),
            pltpu.VMEM((ACC_ROWS * DE,), jnp.float32),
            pltpu.SMEM((8,), jnp.int32),
            pltpu.SemaphoreType.DMA,
        ],
    )


def _dense_body(x_ref, w_ref, b_ref, o_ref):
    o_ref[...] = (jnp.dot(x_ref[...], w_ref[...], preferred_element_type=jnp.float32)
                  + b_ref[...])


def _dense(x, w, b):
    return pl.pallas_call(
        _dense_body,
        out_shape=jax.ShapeDtypeStruct((x.shape[0], w.shape[1]), jnp.float32),
    )(x, w, b[None, :])


def _graph_norm_block(agg, batch2, gnw, gnb, gnms):
    onehot = (batch2 == lax.broadcasted_iota(jnp.int32, (1, G), 1)).astype(jnp.float32)
    counts = jnp.maximum(jnp.sum(onehot, axis=0, keepdims=True), 1.0).reshape(G, 1)
    sums = lax.dot_general(onehot, agg, (((0,), (0,)), ((), ())),
                           preferred_element_type=jnp.float32)
    mean_s = (sums / counts) * gnms
    centered = agg - jnp.dot(onehot, mean_s, preferred_element_type=jnp.float32)
    var = lax.dot_general(onehot, centered * centered, (((0,), (0,)), ((), ())),
                          preferred_element_type=jnp.float32) / counts
    inv = 1.0 / jnp.sqrt(var + 1e-5)
    y = gnw * centered * jnp.dot(onehot, inv, preferred_element_type=jnp.float32) + gnb
    return jnp.where(y >= 0.0, y, 0.01 * y)


def _norm_mid_body(p_ref, a_ref, h_ref, we_ref, batch_ref,
                   gnw_ref, gnb_ref, gnms_ref, wm_ref, bm_ref, o_ref, ohw_ref):
    asum = a_ref[0] + a_ref[1]
    agg = (p_ref[0] + p_ref[1]
           + jnp.dot(asum, we_ref[...], preferred_element_type=jnp.float32)
           + h_ref[...])
    y = _graph_norm_block(agg, batch_ref[...], gnw_ref[...], gnb_ref[...], gnms_ref[...])
    o_ref[...] = y
    ohw_ref[...] = (jnp.dot(y, wm_ref[...], preferred_element_type=jnp.float32)
                    + bm_ref[...])


def _norm_fin_body(p_ref, a_ref, h_ref, we_ref, batch_ref,
                   gnw_ref, gnb_ref, gnms_ref, o_ref):
    asum = a_ref[0] + a_ref[1]
    agg = (p_ref[0] + p_ref[1]
           + jnp.dot(asum, we_ref[...], preferred_element_type=jnp.float32)
           + h_ref[...])
    o_ref[...] = _graph_norm_block(agg, batch_ref[...], gnw_ref[...], gnb_ref[...],
                                   gnms_ref[...])


def kernel(x, index, attr, batch, W_msg0, b_msg0, W_edge0, b_edge0, gn_w0, gn_b0,
           gn_ms0, W_msg1, b_msg1, W_edge1, b_edge1, gn_w1, gn_b1, gn_ms1):
    src = index[0]
    dst = index[1]
    batch2 = batch[:, None]

    attr_flat = attr.reshape(E * DE)
    hw0 = _dense(x, W_msg0, b_msg0 + b_edge0)
    p0, a0 = _sc_consume(True)(hw0, src, dst, attr_flat)
    p0 = p0[:, :N]
    a0 = a0.reshape(NC, NPAD, DE)[:, :N]
    h1, hw1 = pl.pallas_call(
        _norm_mid_body,
        out_shape=(jax.ShapeDtypeStruct((N, D), jnp.float32),
                   jax.ShapeDtypeStruct((N, D), jnp.float32)),
    )(p0, a0, x, W_edge0, batch2, gn_w0[None, :], gn_b0[None, :], gn_ms0[None, :],
      W_msg1, (b_msg1 + b_edge1)[None, :])
    p1 = _sc_consume(False)(hw1, src, dst, attr_flat)[0][:, :N]
    h2 = pl.pallas_call(
        _norm_fin_body,
        out_shape=jax.ShapeDtypeStruct((N, D), jnp.float32),
    )(p1, a0, h1, W_edge1, batch2, gn_w1[None, :], gn_b1[None, :], gn_ms1[None, :])
    return h2


# speedup vs baseline: 1.3793x; 1.0119x over previous
import functools

import jax
import jax.numpy as jnp
from jax import lax
from jax.experimental import pallas as pl
from jax.experimental.pallas import tpu as pltpu
from jax.experimental.pallas import tpu_sc as plsc

N = 10000
E = 320000
D = 128
DE = 16
G = 64

NC = 2
NS = 16
NW = NC * NS
EPW = E // NW
CH = 640
SEG = 64
STAGE = SEG + 16

NPAD = 10240
ROWS_PT = 640
TRASH = ROWS_PT
ACC_ROWS = ROWS_PT + 8


@functools.cache
def _sc_mesh():
    return plsc.VectorSubcoreMesh(core_axis_name="c", subcore_axis_name="s",
                                  num_cores=NC, num_subcores=NS)


def _consume_body(with_attr, hw_hbm, src_hbm, dst_hbm, attr_hbm, pout_hbm,
                  aout_hbm, srcc_v, dstc_v, attrc_v, ssrc_v, sdl_v, sattr_v,
                  rows_v, acc_v, aacc_v, cnt_sm, sem):
    c = lax.axis_index("c")
    me = lax.axis_index("s")
    zf = jnp.zeros((16,), jnp.float32)
    iota16 = lax.iota(jnp.int32, 16)
    trash16 = jnp.zeros((16,), jnp.int32) + TRASH
    zero16i = jnp.zeros((16,), jnp.int32)

    def zrow(i, carry):
        for k8 in range(D // 16):
            acc_v[i, pl.ds(k8 * 16, 16)] = zf
        if with_attr:
            aacc_v[pl.ds(i * DE, 16)] = zf
        return carry

    lax.fori_loop(0, ACC_ROWS, zrow, 0)
    cnt_sm[0] = 0

    def process64():
        pltpu.async_copy(hw_hbm.at[ssrc_v.at[pl.ds(0, SEG)]], rows_v, sem).wait()

        def grp(q, carry3):
            d16 = sdl_v[pl.ds(q * 16, 16)]
            for l in range(16):
                dl = d16[l]
                j = q * 16 + l
                for k8 in range(D // 16):
                    sl = pl.ds(k8 * 16, 16)
                    acc_v[dl, sl] = acc_v[dl, sl] + rows_v[j, sl]
                if with_attr:
                    asl = pl.ds(dl * DE, 16)
                    aacc_v[asl] = aacc_v[asl] + sattr_v[pl.ds(j * DE, 16)]
            return carry3

        lax.fori_loop(0, SEG // 16, grp, 0)

    def pad_stage(cnt):
        for g in range(STAGE // 16):
            posv = iota16 + (g * 16)
            keep = posv < cnt
            sdl_v[pl.ds(g * 16, 16)] = jnp.where(keep, sdl_v[pl.ds(g * 16, 16)],
                                                 trash16)
            ssrc_v[pl.ds(g * 16, 16)] = jnp.where(keep, ssrc_v[pl.ds(g * 16, 16)],
                                                  zero16i)

        if with_attr:
            def zattr(q, carry):
                @pl.when(q >= cnt)
                def _():
                    sattr_v[pl.ds(q * DE, 16)] = zf
                return carry

            lax.fori_loop(0, STAGE, zattr, 0)

    def chunk(k, carry):
        eb = pl.multiple_of(c * (E // NC) + k * CH, 8)
        pltpu.sync_copy(src_hbm.at[pl.ds(eb, CH)], srcc_v)
        pltpu.sync_copy(dst_hbm.at[pl.ds(eb, CH)], dstc_v)
        if with_attr:
            pltpu.sync_copy(attr_hbm.at[pl.ds(eb * DE, CH * DE)], attrc_v)

        def grp(j, carry2):
            d16 = dstc_v[pl.ds(j * 16, 16)]
            s16 = srcc_v[pl.ds(j * 16, 16)]
            o16 = lax.shift_right_logical(d16, 5) & 15
            dl16 = lax.shift_left(lax.shift_right_logical(d16, 9), 5) | (d16 & 31)
            m = o16 == me
            mi = jnp.where(m, 1, 0)
            incl = plsc.cumsum(mi)
            tot = incl[15]

            @pl.when(tot > 0)
            def _():
                r16 = incl - mi
                cnt = cnt_sm[0]
                pos16 = r16 + cnt
                plsc.store_scatter(ssrc_v, [pos16], s16, mask=m)
                plsc.store_scatter(sdl_v, [pos16], dl16, mask=m)
                if with_attr:
                    for col in range(DE):
                        a16 = plsc.load_gather(attrc_v,
                                               [iota16 * DE + (j * 16 * DE + col)])
                        plsc.store_scatter(sattr_v, [pos16 * DE + col], a16,
                                           mask=m)
                cnt2 = cnt + tot
                cnt_sm[0] = cnt2

                @pl.when(cnt2 >= SEG)
                def _():
                    process64()
                    sdl_v[pl.ds(0, 16)] = sdl_v[pl.ds(SEG, 16)]
                    ssrc_v[pl.ds(0, 16)] = ssrc_v[pl.ds(SEG, 16)]
                    if with_attr:
                        for q in range(STAGE - SEG):
                            sattr_v[pl.ds(q * DE, 16)] = sattr_v[
                                pl.ds((q + SEG) * DE, 16)]
                    cnt_sm[0] = cnt2 - SEG
            return carry2

        lax.fori_loop(0, CH // 16, grp, 0)
        return carry

    lax.fori_loop(0, (E // NC) // CH, chunk, 0)

    cnt = cnt_sm[0]
    pad_stage(cnt)
    process64()

    for b in range(ROWS_PT // 32):
        gbase = b << 9
        pltpu.sync_copy(
            acc_v.at[pl.ds(b * 32, 32)],
            pout_hbm.at[c, pl.ds(pl.multiple_of(gbase + me * 32, 32), 32)])
        if with_attr:
            pltpu.sync_copy(
                aacc_v.at[pl.ds(b * 32 * DE, 32 * DE)],
                aout_hbm.at[c,
                            pl.ds(pl.multiple_of((gbase + me * 32) * DE, 32 * DE),
                                  32 * DE)])


@functools.cache
def _sc_consume(with_attr):
    return pl.kernel(
        functools.partial(_consume_body, with_attr),
        out_type=(jax.ShapeDtypeStruct((NC, NPAD, D), jnp.float32),
                  jax.ShapeDtypeStruct((NC, NPAD * DE), jnp.float32)),
        mesh=_sc_mesh(),
        compiler_params=pltpu.CompilerParams(needs_layout_passes=False),
        scratch_types=[
            pltpu.VMEM((CH,), jnp.int32),
            pltpu.VMEM((CH,), jnp.int32),
            pltpu.VMEM((CH * DE,), jnp.float32),
            pltpu.VMEM((STAGE,), jnp.int32),
            pltpu.VMEM((STAGE,), jnp.int32),
            pltpu.VMEM((STAGE * DE,), jnp.float32),
            pltpu.VMEM((SEG, D), jnp.float32),
            pltpu.VMEM((ACC_ROWS, D), jnp.float32),
            pltpu.VMEM((ACC_ROWS * DE,), jnp.float32),
            pltpu.SMEM((8,), jnp.int32),
            pltpu.SemaphoreType.DMA,
        ],
    )


def _dense_body(x_ref, w_ref, b_ref, o_ref):
    o_ref[...] = (jnp.dot(x_ref[...], w_ref[...], preferred_element_type=jnp.float32)
                  + b_ref[...])


def _dense(x, w, b):
    return pl.pallas_call(
        _dense_body,
        out_shape=jax.ShapeDtypeStruct((x.shape[0], w.shape[1]), jnp.float32),
    )(x, w, b[None, :])


def _graph_norm_block(agg, batch2, gnw, gnb, gnms):
    onehot = (batch2 == lax.broadcasted_iota(jnp.int32, (1, G), 1)).astype(jnp.float32)
    counts = jnp.maximum(jnp.sum(onehot, axis=0, keepdims=True), 1.0).reshape(G, 1)
    sums = lax.dot_general(onehot, agg, (((0,), (0,)), ((), ())),
                           preferred_element_type=jnp.float32)
    mean_s = (sums / counts) * gnms
    centered = agg - jnp.dot(onehot, mean_s, preferred_element_type=jnp.float32)
    var = lax.dot_general(onehot, centered * centered, (((0,), (0,)), ((), ())),
                          preferred_element_type=jnp.float32) / counts
    inv = 1.0 / jnp.sqrt(var + 1e-5)
    y = gnw * centered * jnp.dot(onehot, inv, preferred_element_type=jnp.float32) + gnb
    return jnp.where(y >= 0.0, y, 0.01 * y)


def _norm_mid_body(p_ref, a_ref, h_ref, we_ref, batch_ref,
                   gnw_ref, gnb_ref, gnms_ref, wm_ref, bm_ref, o_ref, ohw_ref):
    asum = a_ref[0] + a_ref[1]
    agg = (p_ref[0] + p_ref[1]
           + jnp.dot(asum, we_ref[...], preferred_element_type=jnp.float32)
           + h_ref[...])
    y = _graph_norm_block(agg, batch_ref[...], gnw_ref[...], gnb_ref[...], gnms_ref[...])
    o_ref[...] = y
    ohw_ref[...] = (jnp.dot(y, wm_ref[...], preferred_element_type=jnp.float32)
                    + bm_ref[...])


def _norm_fin_body(p_ref, a_ref, h_ref, we_ref, batch_ref,
                   gnw_ref, gnb_ref, gnms_ref, o_ref):
    asum = a_ref[0] + a_ref[1]
    agg = (p_ref[0] + p_ref[1]
           + jnp.dot(asum, we_ref[...], preferred_element_type=jnp.float32)
           + h_ref[...])
    o_ref[...] = _graph_norm_block(agg, batch_ref[...], gnw_ref[...], gnb_ref[...],
                                   gnms_ref[...])


def kernel(x, index, attr, batch, W_msg0, b_msg0, W_edge0, b_edge0, gn_w0, gn_b0,
           gn_ms0, W_msg1, b_msg1, W_edge1, b_edge1, gn_w1, gn_b1, gn_ms1):
    src = index[0]
    dst = index[1]
    batch2 = batch[:, None]

    attr_flat = attr.reshape(E * DE)
    hw0 = _dense(x, W_msg0, b_msg0 + b_edge0)
    p0, a0 = _sc_consume(True)(hw0, src, dst, attr_flat)
    p0 = p0[:, :N]
    a0 = a0.reshape(NC, NPAD, DE)[:, :N]
    h1, hw1 = pl.pallas_call(
        _norm_mid_body,
        out_shape=(jax.ShapeDtypeStruct((N, D), jnp.float32),
                   jax.ShapeDtypeStruct((N, D), jnp.float32)),
    )(p0, a0, x, W_edge0, batch2, gn_w0[None, :], gn_b0[None, :], gn_ms0[None, :],
      W_msg1, (b_msg1 + b_edge1)[None, :])
    p1 = _sc_consume(False)(hw1, src, dst, attr_flat)[0][:, :N]
    h2 = pl.pallas_call(
        _norm_fin_body,
        out_shape=jax.ShapeDtypeStruct((N, D), jnp.float32),
    )(p1, a0, h1, W_edge1, batch2, gn_w1[None, :], gn_b1[None, :], gn_ms1[None, :])
    return h2
